# baseline (speedup 1.0000x reference)
import jax
import jax.numpy as jnp
from jax.experimental import pallas as pl
from jax.experimental.pallas import tpu as pltpu

_NBUF = 4
_RT = 256


def _make_body(B, D, S, n_chunks, chunks_per_b):
    def body(labels_ref, x_ref, table_ref, o_ref,
             xbuf, obuf, ebuf, in_sems, out_sems, emb_sem):
        emb_copies = [
            pltpu.make_async_copy(
                table_ref.at[labels_ref[b]], ebuf.at[b], emb_sem)
            for b in range(B)
        ]
        for cp in emb_copies:
            cp.start()

        def in_copy(c, slot):
            return pltpu.make_async_copy(
                x_ref.at[pl.ds(c * _RT, _RT)], xbuf.at[slot],
                in_sems.at[slot])

        def out_copy(c, slot):
            return pltpu.make_async_copy(
                obuf.at[slot], o_ref.at[pl.ds(c * _RT, _RT)],
                out_sems.at[slot])

        for i in range(_NBUF):
            in_copy(i, i).start()
        for cp in emb_copies:
            cp.wait()

        def step(c, carry):
            slot = jax.lax.rem(c, _NBUF)
            in_copy(c, slot).wait()

            @pl.when(c >= _NBUF)
            def _():
                out_copy(c - _NBUF, slot).wait()

            b = c // chunks_per_b
            j = jax.lax.rem(c, chunks_per_b)
            e = ebuf[b, j, :]
            obuf[slot] = xbuf[slot] + e[:, None]
            out_copy(c, slot).start()

            @pl.when(c + _NBUF < n_chunks)
            def _():
                in_copy(c + _NBUF, slot).start()

            return carry

        jax.lax.fori_loop(0, n_chunks, step, 0)
        for k in range(_NBUF):
            c = n_chunks - _NBUF + k
            out_copy(c, jax.lax.rem(jnp.int32(c), _NBUF)).wait()

    return body


def kernel(x, spec_labels, table):
    B, D, S = x.shape
    n_rows = B * D
    n_chunks = n_rows // _RT
    chunks_per_b = D // _RT
    x2 = x.reshape(n_rows, S)
    table3 = table.reshape(table.shape[0], chunks_per_b, _RT)

    out = pl.pallas_call(
        _make_body(B, D, S, n_chunks, chunks_per_b),
        in_specs=[
            pl.BlockSpec(memory_space=pltpu.SMEM),
            pl.BlockSpec(memory_space=pl.ANY),
            pl.BlockSpec(memory_space=pl.ANY),
        ],
        out_specs=pl.BlockSpec(memory_space=pl.ANY),
        out_shape=jax.ShapeDtypeStruct((n_rows, S), x.dtype),
        scratch_shapes=[
            pltpu.VMEM((_NBUF, _RT, S), x.dtype),
            pltpu.VMEM((_NBUF, _RT, S), x.dtype),
            pltpu.VMEM((B, chunks_per_b, _RT), x.dtype),
            pltpu.SemaphoreType.DMA((_NBUF,)),
            pltpu.SemaphoreType.DMA((_NBUF,)),
            pltpu.SemaphoreType.DMA,
        ],
        compiler_params=pltpu.CompilerParams(
            vmem_limit_bytes=64 * 1024 * 1024,
        ),
    )(spec_labels.astype(jnp.int32), x2, table3)
    return out.reshape(B, D, S)


# speedup vs baseline: 1.0727x; 1.0727x over previous
import jax
import jax.numpy as jnp
from jax.experimental import pallas as pl
from jax.experimental.pallas import tpu as pltpu

_RT = 512


def _spec_add_kernel(labels_ref, x_ref, emb_ref, o_ref):
    e = emb_ref[0, 0, 0, :]
    o_ref[...] = x_ref[...] + e[:, None]


def kernel(x, spec_labels, table):
    B, D, S = x.shape
    n_rows = B * D
    cpb = D // _RT
    x2 = x.reshape(n_rows, S)
    table3 = table.reshape(table.shape[0], cpb, 1, _RT)
    grid_spec = pltpu.PrefetchScalarGridSpec(
        num_scalar_prefetch=1,
        grid=(n_rows // _RT,),
        in_specs=[
            pl.BlockSpec((_RT, S), lambda i, labels: (i, 0)),
            pl.BlockSpec((1, 1, 1, _RT), lambda i, labels: (labels[i // cpb], i % cpb, 0, 0)),
        ],
        out_specs=pl.BlockSpec((_RT, S), lambda i, labels: (i, 0)),
    )
    out = pl.pallas_call(
        _spec_add_kernel,
        grid_spec=grid_spec,
        out_shape=jax.ShapeDtypeStruct((n_rows, S), x.dtype),
        compiler_params=pltpu.CompilerParams(
            dimension_semantics=("parallel",),
            vmem_limit_bytes=64 * 1024 * 1024,
        ),
    )(spec_labels.astype(jnp.int32), x2, table3)
    return out.reshape(B, D, S)


# speedup vs baseline: 1.2207x; 1.1379x over previous
import jax
import jax.numpy as jnp
from jax.experimental import pallas as pl
from jax.experimental.pallas import tpu as pltpu

_DT = 512
_NBUF = 4


def _make_body(B, D, S):
    cpb = D // _DT

    def body(labels_ref, x_ref, table_ref, o_ref, ebuf, emb_sem):
        emb_copies = [
            pltpu.make_async_copy(
                table_ref.at[labels_ref[b]], ebuf.at[b], emb_sem)
            for b in range(B)
        ]
        for cp in emb_copies:
            cp.start()
        for cp in emb_copies:
            cp.wait()

        def inner(idx, x_blk, o_blk):
            b, d = idx
            e = ebuf[b, d, :]
            o_blk[...] = x_blk[...] + e[None, :, None]

        pipe = pltpu.emit_pipeline(
            inner,
            grid=(B, cpb),
            in_specs=[
                pl.BlockSpec((1, _DT, S), lambda b, d: (b, d, 0),
                             pipeline_mode=pl.Buffered(buffer_count=_NBUF)),
            ],
            out_specs=[
                pl.BlockSpec((1, _DT, S), lambda b, d: (b, d, 0)),
            ],
            _explicit_indices=True,
        )
        pipe(x_ref, o_ref)

    return body


def kernel(x, spec_labels, table):
    B, D, S = x.shape
    cpb = D // _DT
    table3 = table.reshape(table.shape[0], cpb, _DT)

    return pl.pallas_call(
        _make_body(B, D, S),
        in_specs=[
            pl.BlockSpec(memory_space=pltpu.SMEM),
            pl.BlockSpec(memory_space=pl.ANY),
            pl.BlockSpec(memory_space=pl.ANY),
        ],
        out_specs=pl.BlockSpec(memory_space=pl.ANY),
        out_shape=jax.ShapeDtypeStruct((B, D, S), x.dtype),
        scratch_shapes=[
            pltpu.VMEM((B, cpb, _DT), x.dtype),
            pltpu.SemaphoreType.DMA,
        ],
        compiler_params=pltpu.CompilerParams(
            vmem_limit_bytes=64 * 1024 * 1024,
        ),
    )(spec_labels.astype(jnp.int32), x, table3)
